# baseline (speedup 1.0000x reference)
import jax
import jax.numpy as jnp
from jax import lax
from jax.experimental import pallas as pl
from jax.experimental.pallas import tpu as pltpu

_NA = 4
_NC = 1
_EMB = 512
_ANCHORS_W = (32.0, 45.0, 64.0, 90.0)
_ANCHORS_H = (96.0, 135.0, 192.0, 273.0)
_NB, _NGH, _NGW = 8, 38, 68
_NS = _NGH * _NGW
_NSP = 2688
_BOX_CH = _NA * (_NC + 5)
_OUT_CH = 4 + 1 + _NC + _EMB
_HW = 128
_TW = _OUT_CH - _HW


def _head(xb, aw, ah, stride):
    dx = xb[:, 0:1]
    dy = xb[:, 1:2]
    dw = xb[:, 2:3]
    dh = xb[:, 3:4]
    c0 = xb[:, 4:5]
    c1 = xb[:, 5:6]
    idx = lax.broadcasted_iota(jnp.int32, (_NS, 1), 0)
    px = (idx % _NGW).astype(jnp.float32)
    py = (idx // _NGW).astype(jnp.float32)
    gx = aw * dx + px * stride
    gy = ah * dy + py * stride
    gw = aw * jnp.exp(dw)
    gh = ah * jnp.exp(dh)
    conf = jax.nn.sigmoid(c1 - c0)
    cls = jnp.zeros_like(conf)
    return jnp.concatenate([gx, gy, gw, gh, conf, cls], axis=1)


def _dmas(out_ref, embt_ref, heads_ref, sems, p, b):
    cps = [
        pltpu.make_async_copy(
            embt_ref.at[p, pl.ds(0, _NS), pl.ds(_HW, _TW)],
            out_ref.at[b, a, :, pl.ds(_HW, _TW)],
            sems.at[p, a],
        )
        for a in range(_NA)
    ]
    cps.append(
        pltpu.make_async_copy(
            heads_ref.at[p],
            out_ref.at[b, :, :, pl.ds(0, _HW)],
            sems.at[p, _NA],
        )
    )
    return cps


def _body(stride_ref, box_ref, emb_ref, out_ref, embt_ref, heads_ref, sems):
    b = pl.program_id(0)
    nb = pl.num_programs(0)
    p = b % 2
    stride = stride_ref[0, 0]

    @pl.when(b >= 2)
    def _wait_prev_same_parity():
        for cp in _dmas(out_ref, embt_ref, heads_ref, sems, p, b):
            cp.wait()

    emb = emb_ref[0]
    ssq = jnp.sum(emb * emb, axis=0, keepdims=True)
    inv = 1.0 / jnp.maximum(jnp.sqrt(ssq), 1e-12)
    embn = jnp.pad(emb * inv, ((6, 0), (0, _NSP - _NS)))
    embt = jnp.swapaxes(embn, 0, 1)
    embt_ref[p] = embt

    emb_lo = embt[0:_NS, 0:_HW]
    for a in range(_NA):
        hd = _head(box_ref[0, a], _ANCHORS_W[a], _ANCHORS_H[a], stride)
        heads_ref[p, a] = jnp.pad(hd, ((0, 0), (0, _HW - 6))) + emb_lo

    for cp in _dmas(out_ref, embt_ref, heads_ref, sems, p, b):
        cp.start()

    @pl.when(b == nb - 1)
    def _drain_all():
        for cp in _dmas(out_ref, embt_ref, heads_ref, sems, 1 - p, b):
            cp.wait()
        for cp in _dmas(out_ref, embt_ref, heads_ref, sems, p, b):
            cp.wait()


def kernel(p_cat, img_size):
    nB = p_cat.shape[0]
    xf = p_cat.reshape(nB, _BOX_CH + _EMB, _NS)
    box_t = (
        xf[:, :_BOX_CH, :]
        .reshape(nB, _NA, _NC + 5, _NS)
        .transpose(0, 1, 3, 2)
    )
    x_emb = xf[:, _BOX_CH:, :]
    stride = (jnp.asarray(img_size[0], jnp.float32) / _NGW).reshape(1, 1)

    out = pl.pallas_call(
        _body,
        grid=(nB,),
        in_specs=[
            pl.BlockSpec(memory_space=pltpu.SMEM),
            pl.BlockSpec((1, _NA, _NS, _NC + 5), lambda b: (b, 0, 0, 0)),
            pl.BlockSpec((1, _EMB, _NS), lambda b: (b, 0, 0)),
        ],
        out_specs=pl.BlockSpec(memory_space=pl.ANY),
        out_shape=jax.ShapeDtypeStruct((nB, _NA, _NS, _OUT_CH), jnp.float32),
        scratch_shapes=[
            pltpu.VMEM((2, _NSP, _OUT_CH), jnp.float32),
            pltpu.VMEM((2, _NA, _NS, _HW), jnp.float32),
            pltpu.SemaphoreType.DMA((2, _NA + 1)),
        ],
    )(stride, box_t, x_emb)
    return out.reshape(nB, _NA * _NS, _OUT_CH)


# speedup vs baseline: 2.7454x; 2.7454x over previous
import jax
import jax.numpy as jnp
from jax import lax
from jax.experimental import pallas as pl
from jax.experimental.pallas import tpu as pltpu

_NA = 4
_NC = 1
_EMB = 512
_ANCHORS_W = (32.0, 45.0, 64.0, 90.0)
_ANCHORS_H = (96.0, 135.0, 192.0, 273.0)
_NB, _NGH, _NGW = 8, 38, 68
_NS = _NGH * _NGW
_NSP = 2688
_BOX_CH = _NA * (_NC + 5)
_OUT_CH = 4 + 1 + _NC + _EMB


def _head(xb, aw, ah, stride):
    dx = xb[:, 0:1]
    dy = xb[:, 1:2]
    dw = xb[:, 2:3]
    dh = xb[:, 3:4]
    c0 = xb[:, 4:5]
    c1 = xb[:, 5:6]
    idx = lax.broadcasted_iota(jnp.int32, (_NS, 1), 0)
    px = (idx % _NGW).astype(jnp.float32)
    py = (idx // _NGW).astype(jnp.float32)
    gx = aw * dx + px * stride
    gy = ah * dy + py * stride
    gw = aw * jnp.exp(dw)
    gh = ah * jnp.exp(dh)
    conf = jax.nn.sigmoid(c1 - c0)
    cls = jnp.zeros_like(conf)
    return jnp.concatenate([gx, gy, gw, gh, conf, cls], axis=1)


def _body(stride_ref, box_ref, emb_ref, out_ref):
    h = pl.program_id(1)
    stride = stride_ref[0, 0]

    emb = emb_ref[0]
    ssq = jnp.sum(emb * emb, axis=0, keepdims=True)
    inv = 1.0 / jnp.maximum(jnp.sqrt(ssq), 1e-12)
    embn = jnp.pad(emb * inv, ((6, 0), (0, _NSP - _NS)))
    rows = jnp.swapaxes(embn, 0, 1)[0:_NS, :]

    out_ref[0, 0:_NS] = rows
    out_ref[0, _NS:2 * _NS] = rows

    aw0 = jnp.where(h == 0, _ANCHORS_W[0], _ANCHORS_W[2])
    ah0 = jnp.where(h == 0, _ANCHORS_H[0], _ANCHORS_H[2])
    aw1 = jnp.where(h == 0, _ANCHORS_W[1], _ANCHORS_W[3])
    ah1 = jnp.where(h == 0, _ANCHORS_H[1], _ANCHORS_H[3])
    out_ref[0, 0:_NS, 0:6] = _head(box_ref[0, 0], aw0, ah0, stride)
    out_ref[0, _NS:2 * _NS, 0:6] = _head(box_ref[0, 1], aw1, ah1, stride)


def kernel(p_cat, img_size):
    nB = p_cat.shape[0]
    xf = p_cat.reshape(nB, _BOX_CH + _EMB, _NS)
    box_t = (
        xf[:, :_BOX_CH, :]
        .reshape(nB, _NA, _NC + 5, _NS)
        .transpose(0, 1, 3, 2)
    )
    x_emb = xf[:, _BOX_CH:, :]
    stride = (jnp.asarray(img_size[0], jnp.float32) / _NGW).reshape(1, 1)

    out = pl.pallas_call(
        _body,
        grid=(nB, 2),
        in_specs=[
            pl.BlockSpec(memory_space=pltpu.SMEM),
            pl.BlockSpec((1, 2, _NS, _NC + 5), lambda b, h: (b, h, 0, 0)),
            pl.BlockSpec((1, _EMB, _NS), lambda b, h: (b, 0, 0)),
        ],
        out_specs=pl.BlockSpec((1, 2 * _NS, _OUT_CH), lambda b, h: (b, h, 0)),
        out_shape=jax.ShapeDtypeStruct((nB, _NA * _NS, _OUT_CH), jnp.float32),
    )(stride, box_t, x_emb)
    return out


# speedup vs baseline: 3.0470x; 1.1099x over previous
import jax
import jax.numpy as jnp
from jax import lax
from jax.experimental import pallas as pl
from jax.experimental.pallas import tpu as pltpu

_NA = 4
_NC = 1
_EMB = 512
_ANCHORS_W = (32.0, 45.0, 64.0, 90.0)
_ANCHORS_H = (96.0, 135.0, 192.0, 273.0)
_NB, _NGH, _NGW = 8, 38, 68
_NS = _NGH * _NGW
_NSP = 2688
_BOX_CH = _NA * (_NC + 5)
_OUT_CH = 4 + 1 + _NC + _EMB
_HW = 128
_TW = _OUT_CH - _HW


def _head(xb, aw, ah, stride):
    dx = xb[:, 0:1]
    dy = xb[:, 1:2]
    dw = xb[:, 2:3]
    dh = xb[:, 3:4]
    c0 = xb[:, 4:5]
    c1 = xb[:, 5:6]
    idx = lax.broadcasted_iota(jnp.int32, (_NS, 1), 0)
    px = (idx % _NGW).astype(jnp.float32)
    py = (idx // _NGW).astype(jnp.float32)
    gx = aw * dx + px * stride
    gy = ah * dy + py * stride
    gw = aw * jnp.exp(dw)
    gh = ah * jnp.exp(dh)
    conf = jax.nn.sigmoid(c1 - c0)
    cls = jnp.zeros_like(conf)
    return jnp.concatenate([gx, gy, gw, gh, conf, cls], axis=1)


def _dma_pair(out_ref, embt_ref, heads_ref, sems, p, b, a):
    tail_cp = pltpu.make_async_copy(
        embt_ref.at[p, pl.ds(0, _NS), pl.ds(_HW, _TW)],
        out_ref.at[b, pl.ds(a * _NS, _NS), pl.ds(_HW, _TW)],
        sems.at[p, a],
    )
    head_cp = pltpu.make_async_copy(
        heads_ref.at[p, a],
        out_ref.at[b, pl.ds(a * _NS, _NS), pl.ds(0, _HW)],
        sems.at[p, _NA + a],
    )
    return tail_cp, head_cp


def _body(stride_ref, box_ref, emb_ref, out_ref, embt_ref, heads_ref, sems):
    b = pl.program_id(0)
    nb = pl.num_programs(0)
    p = b % 3
    stride = stride_ref[0, 0]

    @pl.when(b >= 3)
    def _wait_prev_same_parity():
        for a in range(_NA):
            for cp in _dma_pair(out_ref, embt_ref, heads_ref, sems, p, b, a):
                cp.wait()

    emb = emb_ref[0]
    ssq = jnp.sum(emb * emb, axis=0, keepdims=True)
    inv = 1.0 / jnp.maximum(jnp.sqrt(ssq), 1e-12)
    embn = jnp.pad(emb * inv, ((6, 0), (0, _NSP - _NS)))
    embt = jnp.swapaxes(embn, 0, 1)
    embt_ref[p] = embt

    emb_lo = embt[0:_NS, 0:_HW]
    xb24 = box_ref[0]
    for a in range(_NA):
        c = a * (_NC + 5)
        hd = _head(xb24[:, c:c + 6], _ANCHORS_W[a], _ANCHORS_H[a], stride)
        heads_ref[p, a] = jnp.pad(hd, ((0, 0), (0, _HW - 6))) + emb_lo

    for a in range(_NA):
        for cp in _dma_pair(out_ref, embt_ref, heads_ref, sems, p, b, a):
            cp.start()

    @pl.when(b == nb - 1)
    def _drain_all():
        for q in (1, 2, 0):
            for a in range(_NA):
                for cp in _dma_pair(
                    out_ref, embt_ref, heads_ref, sems, (p + q) % 3, b, a
                ):
                    cp.wait()


def kernel(p_cat, img_size):
    nB = p_cat.shape[0]
    xf = p_cat.reshape(nB, _BOX_CH + _EMB, _NS)
    box_t = xf[:, :_BOX_CH, :].transpose(0, 2, 1)
    x_emb = xf[:, _BOX_CH:, :]
    stride = (jnp.asarray(img_size[0], jnp.float32) / _NGW).reshape(1, 1)

    out = pl.pallas_call(
        _body,
        grid=(nB,),
        in_specs=[
            pl.BlockSpec(memory_space=pltpu.SMEM),
            pl.BlockSpec((1, _NS, _BOX_CH), lambda b: (b, 0, 0)),
            pl.BlockSpec((1, _EMB, _NS), lambda b: (b, 0, 0)),
        ],
        out_specs=pl.BlockSpec(memory_space=pl.ANY),
        out_shape=jax.ShapeDtypeStruct((nB, _NA * _NS, _OUT_CH), jnp.float32),
        scratch_shapes=[
            pltpu.VMEM((3, _NSP, _OUT_CH), jnp.float32),
            pltpu.VMEM((3, _NA, _NS, _HW), jnp.float32),
            pltpu.SemaphoreType.DMA((3, 2 * _NA)),
        ],
    )(stride, box_t, x_emb)
    return out


# speedup vs baseline: 3.4172x; 1.1215x over previous
import jax
import jax.numpy as jnp
from jax import lax
from jax.experimental import pallas as pl
from jax.experimental.pallas import tpu as pltpu

_NA = 4
_NC = 1
_EMB = 512
_ANCHORS_W = (32.0, 45.0, 64.0, 90.0)
_ANCHORS_H = (96.0, 135.0, 192.0, 273.0)
_NB, _NGH, _NGW = 8, 38, 68
_NS = _NGH * _NGW
_NSP = 2688
_BOX_CH = _NA * (_NC + 5)
_OUT_CH = 4 + 1 + _NC + _EMB
_HW = 128
_TW = _OUT_CH - _HW


def _head(xb, aw, ah, stride):
    dx = xb[:, 0:1]
    dy = xb[:, 1:2]
    dw = xb[:, 2:3]
    dh = xb[:, 3:4]
    c0 = xb[:, 4:5]
    c1 = xb[:, 5:6]
    idx = lax.broadcasted_iota(jnp.int32, (_NS, 1), 0)
    px = (idx % _NGW).astype(jnp.float32)
    py = (idx // _NGW).astype(jnp.float32)
    gx = aw * dx + px * stride
    gy = ah * dy + py * stride
    gw = aw * jnp.exp(dw)
    gh = ah * jnp.exp(dh)
    conf = jax.nn.sigmoid(c1 - c0)
    cls = jnp.zeros_like(conf)
    return jnp.concatenate([gx, gy, gw, gh, conf, cls], axis=1)


def _dma_pair(out_ref, embt_ref, heads_ref, sems, p, b, a):
    tail_cp = pltpu.make_async_copy(
        embt_ref.at[p, pl.ds(0, _NS), pl.ds(_HW, _TW)],
        out_ref.at[b, pl.ds(a * _NS, _NS), pl.ds(_HW, _TW)],
        sems.at[p, a],
    )
    head_cp = pltpu.make_async_copy(
        heads_ref.at[p, a],
        out_ref.at[b, pl.ds(a * _NS, _NS), pl.ds(0, _HW)],
        sems.at[p, _NA + a],
    )
    return tail_cp, head_cp


def _body(stride_ref, x_ref, out_ref, embt_ref, heads_ref, sems):
    b = pl.program_id(0)
    nb = pl.num_programs(0)
    p = b % 3
    stride = stride_ref[0, 0]

    @pl.when(b >= 3)
    def _wait_prev_same_parity():
        for a in range(_NA):
            for cp in _dma_pair(out_ref, embt_ref, heads_ref, sems, p, b, a):
                cp.wait()

    emb = x_ref[0, _BOX_CH:, :]
    ssq = jnp.sum(emb * emb, axis=0, keepdims=True)
    inv = 1.0 / jnp.maximum(jnp.sqrt(ssq), 1e-12)
    embn = jnp.pad(emb * inv, ((6, 0), (0, _NSP - _NS)))
    embt = jnp.swapaxes(embn, 0, 1)
    embt_ref[p] = embt

    emb_lo = embt[0:_NS, 0:_HW]
    box_nat = jnp.pad(x_ref[0, 0:_BOX_CH, :], ((0, 0), (0, _NSP - _NS)))
    xb24 = jnp.swapaxes(box_nat, 0, 1)[0:_NS]
    for a in range(_NA):
        c = a * (_NC + 5)
        hd = _head(xb24[:, c:c + 6], _ANCHORS_W[a], _ANCHORS_H[a], stride)
        heads_ref[p, a] = jnp.pad(hd, ((0, 0), (0, _HW - 6))) + emb_lo

    for a in range(_NA):
        for cp in _dma_pair(out_ref, embt_ref, heads_ref, sems, p, b, a):
            cp.start()

    @pl.when(b == nb - 1)
    def _drain_all():
        for q in (1, 2, 0):
            for a in range(_NA):
                for cp in _dma_pair(
                    out_ref, embt_ref, heads_ref, sems, (p + q) % 3, b, a
                ):
                    cp.wait()


def kernel(p_cat, img_size):
    nB = p_cat.shape[0]
    xf = p_cat.reshape(nB, _BOX_CH + _EMB, _NS)
    stride = (jnp.asarray(img_size[0], jnp.float32) / _NGW).reshape(1, 1)

    out = pl.pallas_call(
        _body,
        grid=(nB,),
        in_specs=[
            pl.BlockSpec(memory_space=pltpu.SMEM),
            pl.BlockSpec((1, _BOX_CH + _EMB, _NS), lambda b: (b, 0, 0)),
        ],
        out_specs=pl.BlockSpec(memory_space=pl.ANY),
        out_shape=jax.ShapeDtypeStruct((nB, _NA * _NS, _OUT_CH), jnp.float32),
        scratch_shapes=[
            pltpu.VMEM((3, _NSP, _OUT_CH), jnp.float32),
            pltpu.VMEM((3, _NA, _NS, _HW), jnp.float32),
            pltpu.SemaphoreType.DMA((3, 2 * _NA)),
        ],
    )(stride, xf)
    return out
